# baseline (speedup 1.0000x reference)
import functools
import math

import numpy as np
import jax
import jax.numpy as jnp
from jax.experimental import pallas as pl

_H = 16
_HI = 4
_DIDX = 32
_KSEL = 512
_BM = 256

def _dotT(a, b):
    return jax.lax.dot_general(a.astype(jnp.bfloat16), b.astype(jnp.bfloat16),
                               (((1,), (1,)), ((), ())),
                               preferred_element_type=jnp.float32)


def _rope_tables(T, dh):
    base = 10000.0
    inv = 1.0 / (base ** (np.arange(0, dh, 2, dtype=np.float64) / dh))
    wavelen = 2.0 * np.pi / inv
    ramp = np.clip((wavelen - 1.0) / (32.0 - 1.0), 0.0, 1.0)
    scale = 1.0 + (32.0 - 1.0) * ramp
    t = np.arange(T, dtype=np.float64)[:, None] / scale[None, :]
    freqs = t * inv[None, :]
    emb = np.concatenate([freqs, freqs], axis=-1)
    cos = np.cos(emb).astype(np.float32)
    sin = np.sin(emb).astype(np.float32)
    ce = cos[:, ::2]
    se = sin[:, ::2]
    cf1 = np.concatenate([ce, ce], axis=1)
    sf1 = np.concatenate([-se, se], axis=1)
    CF = np.tile(cf1, (1, _H))
    SF = np.tile(sf1, (1, _H))
    return jnp.asarray(CF), jnp.asarray(SF)


def _proj_kernel(x_ref, w_ref, cf_ref, sf_ref,
                 q_ref, k_ref, v_ref, go_ref, qiw_ref, wlf_ref, kip_ref,
                 *, C, qscale):
    y = _dotT(x_ref[...], w_ref[...])
    cf = cf_ref[...]
    sf = sf_ref[...]
    bf16 = jnp.bfloat16
    q_ref[...] = ((y[:, 0:C] * cf + y[:, C:2 * C] * sf)
                  * jnp.float32(qscale)).astype(bf16)
    k_ref[...] = (y[:, 2 * C:3 * C] * cf + y[:, 3 * C:4 * C] * sf).astype(bf16)
    v_ref[...] = (y[:, 4 * C:5 * C] * jax.nn.sigmoid(y[:, 5 * C:6 * C])).astype(bf16)
    go_ref[...] = jax.nn.sigmoid(y[:, 6 * C:7 * C]).astype(bf16)
    qiw_ref[...] = y[:, 7 * C:7 * C + 256].astype(bf16)
    wlf_ref[...] = y[:, 7 * C + 128:7 * C + 256]
    kip_ref[...] = y[:, 7 * C + 256:7 * C + 384].astype(bf16)


def _attn_kernel(q_ref, go_ref, qiw_ref, wlf_ref, kip_ref, k_ref, v_ref,
                 wo_ref, bias_ref, y_ref, *, W, dh, kk, row_off, do_topk,
                 mbits):
    blk = pl.program_id(0)
    rows = row_off + blk * _BM + jax.lax.broadcasted_iota(jnp.int32, (_BM, W), 0)
    cols = jax.lax.broadcasted_iota(jnp.int32, (_BM, W), 1)
    causal = cols <= rows

    if do_topk:
        qiw = qiw_ref[...]
        kI = kip_ref[...][:, 0:_DIDX]
        wl = wlf_ref[...][:, 0:_HI]
        w = jax.nn.sigmoid(wl + bias_ref[0:1, 0:_HI])

        scores = None
        for h in range(_HI):
            qih = qiw[:, h * _DIDX:(h + 1) * _DIDX]
            lg = _dotT(qih, kI)
            contrib = w[:, h:h + 1] * jnp.maximum(lg, 0.0)
            scores = contrib if scores is None else scores + contrib

        sm = jnp.where(causal, scores, jnp.float32(-jnp.inf))

        u = jax.lax.bitcast_convert_type(sm, jnp.uint32)
        sign = jnp.uint32(0x80000000)
        ukey = jnp.where(u >= sign, ~u, u | sign)

        one = jnp.float32(1.0)
        zero = jnp.float32(0.0)
        kkf = jnp.float32(kk)
        cand = jnp.full((_BM, 1), sign, jnp.uint32)
        for b in range(30, -1, -1):
            t_ = cand | jnp.uint32(1 << b)
            cnt = jnp.sum(jnp.where(ukey >= t_, one, zero), axis=1,
                          keepdims=True)
            cand = jnp.where(cnt >= kkf, t_, cand)

        gt = ukey > cand
        n_gt = jnp.sum(jnp.where(gt, one, zero), axis=1, keepdims=True)
        need = kkf - n_gt
        ties = ukey == cand
        mcand = jnp.zeros((_BM, 1), jnp.int32)
        for b in range(mbits - 1, -1, -1):
            t_ = mcand | jnp.int32(1 << b)
            cnt = jnp.sum(jnp.where(ties & (cols < t_), one, zero), axis=1,
                          keepdims=True)
            mcand = jnp.where(cnt <= need, t_, mcand)
        mask = (gt | (ties & (cols < mcand))) & causal
    else:
        mask = causal

    mb = jnp.where(mask, jnp.float32(0.0), jnp.float32(-1e30))

    qb = q_ref[...]
    kf = k_ref[...]
    vf = v_ref[...]
    outs = []
    for h in range(_H):
        qh = qb[:, h * dh:(h + 1) * dh]
        kh = kf[:, h * dh:(h + 1) * dh]
        vh = vf[:, h * dh:(h + 1) * dh]
        att = _dotT(qh, kh) + mb
        p = jnp.exp2(att)
        l_ = jnp.sum(p, axis=1, keepdims=True)
        oh = jax.lax.dot_general(p.astype(jnp.bfloat16), vh,
                                 (((1,), (0,)), ((), ())),
                                 preferred_element_type=jnp.float32)
        outs.append(oh * (jnp.float32(1.0) / l_))
    out = jnp.concatenate(outs, axis=1) * go_ref[...]
    y_ref[...] = _dotT(out, wo_ref[...])


def kernel(x, W_Iq, W_Ik, W_Iw, gate_bias, W_q, W_k, W_v, W_gv, W_go, W_o):
    b, T, C = x.shape
    dh = C // _H
    kk = min(_KSEL, T)
    x2 = x.reshape(T, C).astype(jnp.bfloat16)

    CF, SF = _rope_tables(T, dh)

    def _deint(Wm):
        w4 = Wm.astype(jnp.bfloat16).reshape(_H, dh // 2, 2, C).transpose(0, 2, 1, 3)
        we = w4[:, 0]
        wo = w4[:, 1]
        P = jnp.concatenate([we, wo], axis=1).reshape(C, C)
        Q = jnp.concatenate([wo, we], axis=1).reshape(C, C)
        return P, Q

    WqP, WqQ = _deint(W_q)
    WkP, WkQ = _deint(W_k)

    f32 = jnp.float32
    bf16 = jnp.bfloat16
    Wcat = jnp.concatenate([
        WqP, WqQ, WkP, WkQ,
        W_v.astype(bf16), W_gv.astype(bf16), W_go.astype(bf16),
        W_Iq.astype(bf16),
        W_Iw.astype(bf16),
        jnp.zeros((124, C), bf16),
        W_Ik.astype(bf16),
        jnp.zeros((96, C), bf16),
    ], axis=0)
    NCAT = 7 * C + 384

    nb = T // _BM
    blk_row = lambda i: (i, 0)
    full = lambda i: (0, 0)
    qscale = (1.0 / math.sqrt(dh)) * math.log2(math.e)

    q, k, v, go, qiw, wlf, kip = pl.pallas_call(
        functools.partial(_proj_kernel, C=C, qscale=qscale),
        grid=(nb,),
        in_specs=[
            pl.BlockSpec((_BM, C), blk_row),
            pl.BlockSpec((NCAT, C), full),
            pl.BlockSpec((_BM, C), blk_row),
            pl.BlockSpec((_BM, C), blk_row),
        ],
        out_specs=[
            pl.BlockSpec((_BM, C), blk_row),
            pl.BlockSpec((_BM, C), blk_row),
            pl.BlockSpec((_BM, C), blk_row),
            pl.BlockSpec((_BM, C), blk_row),
            pl.BlockSpec((_BM, 256), blk_row),
            pl.BlockSpec((_BM, 128), blk_row),
            pl.BlockSpec((_BM, 128), blk_row),
        ],
        out_shape=[
            jax.ShapeDtypeStruct((T, C), bf16),
            jax.ShapeDtypeStruct((T, C), bf16),
            jax.ShapeDtypeStruct((T, C), bf16),
            jax.ShapeDtypeStruct((T, C), bf16),
            jax.ShapeDtypeStruct((T, 256), bf16),
            jax.ShapeDtypeStruct((T, 128), f32),
            jax.ShapeDtypeStruct((T, 128), bf16),
        ],
    )(x2, Wcat, CF, SF)

    biasp = jnp.zeros((8, 128), f32).at[0, :_HI].set(gate_bias)
    Wo16 = W_o.astype(bf16)

    groups = [
        (0, 2, 512, False, 0),
        (512, 2, 1024, True, 11),
        (1024, 2, 1536, True, 11),
        (1536, 2, 2048, True, 12),
    ]
    ys = []
    for row_off, nblk, W, do_topk, mbits in groups:
        boff = row_off // _BM
        rows_map = lambda i, boff=boff: (i + boff, 0)
        y = pl.pallas_call(
            functools.partial(_attn_kernel, W=W, dh=dh, kk=kk,
                              row_off=row_off, do_topk=do_topk, mbits=mbits),
            grid=(nblk,),
            in_specs=[
                pl.BlockSpec((_BM, C), rows_map),
                pl.BlockSpec((_BM, C), rows_map),
                pl.BlockSpec((_BM, 256), rows_map),
                pl.BlockSpec((_BM, 128), rows_map),
                pl.BlockSpec((W, 128), full),
                pl.BlockSpec((W, C), full),
                pl.BlockSpec((W, C), full),
                pl.BlockSpec((C, C), full),
                pl.BlockSpec((8, 128), full),
            ],
            out_specs=pl.BlockSpec((_BM, C), blk_row),
            out_shape=jax.ShapeDtypeStruct((nblk * _BM, C), f32),
        )(q, go, qiw, wlf, kip, k, v, Wo16, biasp)
        ys.append(y)

    return jnp.concatenate(ys, axis=0).reshape(b, T, C)


# speedup vs baseline: 1.0530x; 1.0530x over previous
import functools
import math

import numpy as np
import jax
import jax.numpy as jnp
from jax.experimental import pallas as pl

_H = 16
_HI = 4
_DIDX = 32
_KSEL = 512
_BM = 256

def _dotT(a, b):
    return jax.lax.dot_general(a.astype(jnp.bfloat16), b.astype(jnp.bfloat16),
                               (((1,), (1,)), ((), ())),
                               preferred_element_type=jnp.float32)


def _rope_tables(T, dh):
    base = 10000.0
    inv = 1.0 / (base ** (np.arange(0, dh, 2, dtype=np.float64) / dh))
    wavelen = 2.0 * np.pi / inv
    ramp = np.clip((wavelen - 1.0) / (32.0 - 1.0), 0.0, 1.0)
    scale = 1.0 + (32.0 - 1.0) * ramp
    t = np.arange(T, dtype=np.float64)[:, None] / scale[None, :]
    freqs = t * inv[None, :]
    emb = np.concatenate([freqs, freqs], axis=-1)
    cos = np.cos(emb).astype(np.float32)
    sin = np.sin(emb).astype(np.float32)
    ce = cos[:, ::2]
    se = sin[:, ::2]
    cf1 = np.concatenate([ce, ce], axis=1)
    sf1 = np.concatenate([-se, se], axis=1)
    CF = np.tile(cf1, (1, _H))
    SF = np.tile(sf1, (1, _H))
    return jnp.asarray(CF), jnp.asarray(SF)


def _proj_kernel(x_ref, w_ref, cf_ref, sf_ref,
                 q_ref, k_ref, v_ref, go_ref, qiw_ref, wlf_ref, kip_ref,
                 *, C, qscale):
    y = _dotT(x_ref[...], w_ref[...])
    cf = cf_ref[...]
    sf = sf_ref[...]
    bf16 = jnp.bfloat16
    q_ref[...] = ((y[:, 0:C] * cf + y[:, C:2 * C] * sf)
                  * jnp.float32(qscale)).astype(bf16)
    k_ref[...] = (y[:, 2 * C:3 * C] * cf + y[:, 3 * C:4 * C] * sf).astype(bf16)
    v_ref[...] = (y[:, 4 * C:5 * C] * jax.nn.sigmoid(y[:, 5 * C:6 * C])).astype(bf16)
    go_ref[...] = jax.nn.sigmoid(y[:, 6 * C:7 * C]).astype(bf16)
    qiw_ref[...] = y[:, 7 * C:7 * C + 256].astype(bf16)
    wlf_ref[...] = y[:, 7 * C + 128:7 * C + 256]
    kip_ref[...] = y[:, 7 * C + 256:7 * C + 384].astype(bf16)


def _attn_kernel(q_ref, go_ref, qiw_ref, wlf_ref, kip_ref, k_ref, v_ref,
                 wo_ref, bias_ref, y_ref, *, W, dh, kk, row_off, do_topk,
                 mbits):
    blk = pl.program_id(0)
    rows = row_off + blk * _BM + jax.lax.broadcasted_iota(jnp.int32, (_BM, W), 0)
    cols = jax.lax.broadcasted_iota(jnp.int32, (_BM, W), 1)
    causal = cols <= rows

    if do_topk:
        qiw = qiw_ref[...]
        kI = kip_ref[...][:, 0:_DIDX]
        wl = wlf_ref[...][:, 0:_HI]
        w = jax.nn.sigmoid(wl + bias_ref[0:1, 0:_HI])

        scores = None
        for h in range(_HI):
            qih = qiw[:, h * _DIDX:(h + 1) * _DIDX]
            lg = _dotT(qih, kI)
            contrib = w[:, h:h + 1] * jnp.maximum(lg, 0.0)
            scores = contrib if scores is None else scores + contrib

        sm = jnp.where(causal, scores, jnp.float32(-jnp.inf))

        u = jax.lax.bitcast_convert_type(sm, jnp.uint32)
        sign = jnp.uint32(0x80000000)
        ukey = jnp.where(u >= sign, ~u, u | sign)

        one = jnp.float32(1.0)
        zero = jnp.float32(0.0)
        kkf = jnp.float32(kk)
        cand = jnp.full((_BM, 1), sign, jnp.uint32)
        for b in range(30, -1, -1):
            t_ = cand | jnp.uint32(1 << b)
            cnt = jnp.sum(jnp.where(ukey >= t_, one, zero), axis=1,
                          keepdims=True)
            cand = jnp.where(cnt >= kkf, t_, cand)

        gt = ukey > cand
        n_gt = jnp.sum(jnp.where(gt, one, zero), axis=1, keepdims=True)
        need = kkf - n_gt
        ties = ukey == cand
        mcand = jnp.zeros((_BM, 1), jnp.int32)
        for b in range(mbits - 1, -1, -1):
            t_ = mcand | jnp.int32(1 << b)
            cnt = jnp.sum(jnp.where(ties & (cols < t_), one, zero), axis=1,
                          keepdims=True)
            mcand = jnp.where(cnt <= need, t_, mcand)
        mask = (gt | (ties & (cols < mcand))) & causal
    else:
        mask = causal

    mb = jnp.where(mask, jnp.float32(0.0), jnp.float32(-1e30))

    qb = q_ref[...]
    kf = k_ref[...]
    vf = v_ref[...]
    outs = []
    for h in range(_H):
        qh = qb[:, h * dh:(h + 1) * dh]
        kh = kf[:, h * dh:(h + 1) * dh]
        vh = vf[:, h * dh:(h + 1) * dh]
        att = _dotT(qh, kh) + mb
        p = jnp.exp2(att)
        l_ = jnp.sum(p, axis=1, keepdims=True)
        oh = jax.lax.dot_general(p.astype(jnp.bfloat16), vh,
                                 (((1,), (0,)), ((), ())),
                                 preferred_element_type=jnp.float32)
        outs.append(oh * (jnp.float32(1.0) / l_))
    out = jnp.concatenate(outs, axis=1) * go_ref[...]
    y_ref[...] = _dotT(out, wo_ref[...])


def kernel(x, W_Iq, W_Ik, W_Iw, gate_bias, W_q, W_k, W_v, W_gv, W_go, W_o):
    b, T, C = x.shape
    dh = C // _H
    kk = min(_KSEL, T)
    x2 = x.reshape(T, C).astype(jnp.bfloat16)

    CF, SF = _rope_tables(T, dh)

    def _deint(Wm):
        w4 = Wm.reshape(_H, dh // 2, 2, C).transpose(0, 2, 1, 3)
        we = w4[:, 0]
        wo = w4[:, 1]
        P = jnp.concatenate([we, wo], axis=1).reshape(C, C)
        Q = jnp.concatenate([wo, we], axis=1).reshape(C, C)
        return P, Q

    WqP, WqQ = _deint(W_q)
    WkP, WkQ = _deint(W_k)

    f32 = jnp.float32
    bf16 = jnp.bfloat16
    Wcat = jnp.concatenate([
        WqP, WqQ, WkP, WkQ, W_v, W_gv, W_go,
        W_Iq,
        W_Iw,
        jnp.zeros((124, C), f32),
        W_Ik,
        jnp.zeros((96, C), f32),
    ], axis=0).astype(bf16)
    NCAT = 7 * C + 384

    nb = T // _BM
    blk_row = lambda i: (i, 0)
    full = lambda i: (0, 0)
    qscale = (1.0 / math.sqrt(dh)) * math.log2(math.e)

    q, k, v, go, qiw, wlf, kip = pl.pallas_call(
        functools.partial(_proj_kernel, C=C, qscale=qscale),
        grid=(nb,),
        in_specs=[
            pl.BlockSpec((_BM, C), blk_row),
            pl.BlockSpec((NCAT, C), full),
            pl.BlockSpec((_BM, C), blk_row),
            pl.BlockSpec((_BM, C), blk_row),
        ],
        out_specs=[
            pl.BlockSpec((_BM, C), blk_row),
            pl.BlockSpec((_BM, C), blk_row),
            pl.BlockSpec((_BM, C), blk_row),
            pl.BlockSpec((_BM, C), blk_row),
            pl.BlockSpec((_BM, 256), blk_row),
            pl.BlockSpec((_BM, 128), blk_row),
            pl.BlockSpec((_BM, 128), blk_row),
        ],
        out_shape=[
            jax.ShapeDtypeStruct((T, C), bf16),
            jax.ShapeDtypeStruct((T, C), bf16),
            jax.ShapeDtypeStruct((T, C), bf16),
            jax.ShapeDtypeStruct((T, C), bf16),
            jax.ShapeDtypeStruct((T, 256), bf16),
            jax.ShapeDtypeStruct((T, 128), f32),
            jax.ShapeDtypeStruct((T, 128), bf16),
        ],
    )(x2, Wcat, CF, SF)

    biasp = jnp.zeros((8, 128), f32).at[0, :_HI].set(gate_bias)
    Wo16 = W_o.astype(bf16)

    groups = [
        (0, 2, 512, False, 0),
        (512, 2, 1024, True, 11),
        (1024, 2, 1536, True, 11),
        (1536, 2, 2048, True, 12),
    ]
    ys = []
    for row_off, nblk, W, do_topk, mbits in groups:
        boff = row_off // _BM
        rows_map = lambda i, boff=boff: (i + boff, 0)
        y = pl.pallas_call(
            functools.partial(_attn_kernel, W=W, dh=dh, kk=kk,
                              row_off=row_off, do_topk=do_topk, mbits=mbits),
            grid=(nblk,),
            in_specs=[
                pl.BlockSpec((_BM, C), rows_map),
                pl.BlockSpec((_BM, C), rows_map),
                pl.BlockSpec((_BM, 256), rows_map),
                pl.BlockSpec((_BM, 128), rows_map),
                pl.BlockSpec((W, 128), full),
                pl.BlockSpec((W, C), full),
                pl.BlockSpec((W, C), full),
                pl.BlockSpec((C, C), full),
                pl.BlockSpec((8, 128), full),
            ],
            out_specs=pl.BlockSpec((_BM, C), blk_row),
            out_shape=jax.ShapeDtypeStruct((nblk * _BM, C), f32),
        )(q, go, qiw, wlf, kip, k, v, Wo16, biasp)
        ys.append(y)

    return jnp.concatenate(ys, axis=0).reshape(b, T, C)
